# baseline (speedup 1.0000x reference)
import jax
import jax.numpy as jnp
from jax import lax
from jax.experimental import pallas as pl
from jax.experimental.pallas import tpu as pltpu
from jax.experimental.pallas import tpu_sc as plsc

B, L, H = 1024, 200, 3
D = 32
N = B * L
NC, NS, LANES = 2, 16, 16
NW = NC * NS
ROWS_W = N // NW
CHUNK = 640
NIT = ROWS_W // CHUNK
GSL = 128
NG = CHUNK // GSL


def _body(idx0_hbm, idx1_hbm, idx2_hbm, feats_hbm, t0, t1, t2, wb_hbm,
          out_hbm, idx0_v, idx1_v, idx2_v, feats_v, wb_v,
          g0_v, g1_v, g2_v, e_v, sem):
    wid = lax.axis_index("s") * NC + lax.axis_index("c")

    pltpu.sync_copy(wb_hbm, wb_v)
    w_lo = wb_v[pl.ds(0, LANES)]
    w_hi = wb_v[pl.ds(LANES, LANES)]
    b_lo = wb_v[pl.ds(2 * LANES, LANES)]
    b_hi = wb_v[pl.ds(3 * LANES, LANES)]

    def iteration(it, carry):
        base = wid * ROWS_W + it * CHUNK

        pltpu.sync_copy(feats_hbm.at[pl.ds(base, CHUNK)], feats_v)
        pltpu.sync_copy(idx0_hbm.at[pl.ds(base, CHUNK)], idx0_v)
        pltpu.sync_copy(idx1_hbm.at[pl.ds(base, CHUNK)], idx1_v)
        pltpu.sync_copy(idx2_hbm.at[pl.ds(base, CHUNK)], idx2_v)

        copies = []
        for tbl, iv, gv in ((t0, idx0_v, g0_v), (t1, idx1_v, g1_v),
                            (t2, idx2_v, g2_v)):
            for j in range(NG):
                copies.append(pltpu.make_async_copy(
                    tbl.at[iv.at[pl.ds(j * GSL, GSL)]],
                    gv.at[pl.ds(j * GSL, GSL), :],
                    sem))
        for c in copies:
            c.start()

        def enc(i16, c):
            fvec = feats_v[pl.ds(i16 * LANES, LANES)]
            for k in range(LANES):
                fv = jnp.full((LANES,), fvec[k])
                e_v[i16 * LANES + k, pl.ds(0, LANES)] = fv * w_lo + b_lo
                e_v[i16 * LANES + k, pl.ds(LANES, LANES)] = fv * w_hi + b_hi
            return c
        lax.fori_loop(0, CHUNK // LANES, enc, 0)

        for c in copies:
            c.wait()

        for s, gv in enumerate((g0_v, g1_v, g2_v, e_v)):
            pltpu.sync_copy(gv, out_hbm.at[pl.ds(base, CHUNK), s, :])
        return carry

    lax.fori_loop(0, NIT, iteration, 0)


@jax.jit
def _sc_embed(idx0, idx1, idx2, feats, t0, t1, t2, wb):
    mesh = plsc.VectorSubcoreMesh(core_axis_name="c", subcore_axis_name="s",
                                  num_cores=NC, num_subcores=NS)
    f = pl.kernel(
        _body,
        out_type=jax.ShapeDtypeStruct((N, H + 1, D), jnp.float32),
        mesh=mesh,
        compiler_params=pltpu.CompilerParams(use_tc_tiling_on_sc=False),
        scratch_types=[
            pltpu.VMEM((CHUNK,), jnp.int32),
            pltpu.VMEM((CHUNK,), jnp.int32),
            pltpu.VMEM((CHUNK,), jnp.int32),
            pltpu.VMEM((CHUNK,), jnp.float32),
            pltpu.VMEM((4 * LANES,), jnp.float32),
            pltpu.VMEM((CHUNK, D), jnp.float32),
            pltpu.VMEM((CHUNK, D), jnp.float32),
            pltpu.VMEM((CHUNK, D), jnp.float32),
            pltpu.VMEM((CHUNK, D), jnp.float32),
            pltpu.SemaphoreType.DMA,
        ],
    )
    return f(idx0, idx1, idx2, feats, t0, t1, t2, wb)


def kernel(tokens, features, T0, T1, T2, W, b):
    tok = tokens.reshape(N, H)
    feats = features.reshape(N)
    wb = jnp.concatenate([W.reshape(D), b.reshape(D)])
    out = _sc_embed(tok[:, 0], tok[:, 1], tok[:, 2], feats, T0, T1, T2, wb)
    return out.reshape(B, L, (H + 1) * D)
